# Initial kernel scaffold; baseline (speedup 1.0000x reference)
import jax
import jax.numpy as jnp
from jax.experimental import pallas as pl


def kernel(node_feat, edge_feat, indegree, n2p_idx, e2p_idx, pool_seg, weight, bias, W0, b0, W1, b1, Wm, bm):
    raise NotImplementedError("write your pallas kernel here")


# speedup vs baseline: 13.6169x; 13.6169x over previous
import functools

import jax
import jax.numpy as jnp
from jax import lax
from jax.experimental import pallas as pl
from jax.experimental.pallas import tpu as pltpu
from jax.experimental.pallas import tpu_sc as plsc

_N = 10000
_P = 100000
_LL = 16
_IN = 16
_H = 256

_NC = 2
_NS = 16
_NW = _NC * _NS

_CPOOL = 128
_NCHUNK = 782
_PP = _NCHUNK * _CPOOL
_FLAT = _CPOOL * _LL


def _gather_body(node_hbm, edge_hbm, n2pt_hbm, e2pt_hbm, out_hbm,
                 nidx_v, eidx_v, nbuf_v, ebuf_v, wide_v, nsem, esem):
    wid = lax.axis_index("s") * _NC + lax.axis_index("c")

    def chunk_body(t, carry):
        c = wid + _NW * t

        @pl.when(c < _NCHUNK)
        def _():
            ib = _CPOOL * c
            pltpu.sync_copy(n2pt_hbm.at[:, pl.ds(ib, _CPOOL)], nidx_v)
            pltpu.sync_copy(e2pt_hbm.at[:, pl.ds(ib, _CPOOL)], eidx_v)
            copies = []
            for a in range(_LL):
                copies.append(pltpu.async_copy(
                    node_hbm.at[nidx_v.at[a]],
                    nbuf_v.at[pl.ds(_CPOOL * a, _CPOOL)], nsem))
                copies.append(pltpu.async_copy(
                    edge_hbm.at[eidx_v.at[a]],
                    ebuf_v.at[pl.ds(_CPOOL * a, _CPOOL)], esem))
            for cp in copies:
                cp.wait()

            def row_body(r, carry2):
                for a in range(_LL):
                    wide_v[r, pl.ds(_IN * a, _IN)] = (
                        nbuf_v[_CPOOL * a + r, :] + ebuf_v[_CPOOL * a + r, :])
                return carry2

            lax.fori_loop(0, _CPOOL, row_body, 0)
            pltpu.sync_copy(wide_v, out_hbm.at[pl.ds(ib, _CPOOL)])

        return carry

    nt = (_NCHUNK + _NW - 1) // _NW
    lax.fori_loop(0, nt, chunk_body, 0)


def _gather_perm(node_feat, edge_feat, n2pt, e2pt):
    mesh = plsc.VectorSubcoreMesh(core_axis_name="c", subcore_axis_name="s",
                                  num_cores=_NC, num_subcores=_NS)
    kern = functools.partial(
        pl.kernel,
        out_type=jax.ShapeDtypeStruct((_PP, _LL * _IN), jnp.float32),
        mesh=mesh,
        compiler_params=pltpu.CompilerParams(use_tc_tiling_on_sc=False),
        scratch_types=[
            pltpu.VMEM((_LL, _CPOOL), jnp.int32),
            pltpu.VMEM((_LL, _CPOOL), jnp.int32),
            pltpu.VMEM((_FLAT, _IN), jnp.float32),
            pltpu.VMEM((_FLAT, _IN), jnp.float32),
            pltpu.VMEM((_CPOOL, _LL * _IN), jnp.float32),
            pltpu.SemaphoreType.DMA,
            pltpu.SemaphoreType.DMA,
        ],
    )(_gather_body)
    return kern(node_feat, edge_feat, n2pt, e2pt)


_BP = 2176


def _mm_body(p_ref, w_ref, b_ref, o_ref):
    acc = jnp.dot(p_ref[...], w_ref[...], preferred_element_type=jnp.float32)
    o_ref[...] = jnp.maximum(acc + b_ref[...], 0.0)


def _matmul_act(perm, w2, bias):
    return pl.pallas_call(
        _mm_body,
        grid=(_PP // _BP,),
        in_specs=[
            pl.BlockSpec((_BP, _LL * _IN), lambda i: (i, 0)),
            pl.BlockSpec((_LL * _IN, _H), lambda i: (0, 0)),
            pl.BlockSpec((1, _H), lambda i: (0, 0)),
        ],
        out_specs=pl.BlockSpec((_BP, _H), lambda i: (i, 0)),
        out_shape=jax.ShapeDtypeStruct((_PP, _H), jnp.float32),
    )(perm, w2, bias.reshape(1, _H))


_RC = 128
_NPW = 320


def _segsum_body(act_hbm, seg_hbm, bounds_hbm, zeros_hbm, out_hbm,
                 rows_v, seg_v, bounds_v, acc_v):
    wid = lax.axis_index("s") * _NC + lax.axis_index("c")
    pltpu.sync_copy(bounds_hbm, bounds_v)
    pltpu.sync_copy(zeros_hbm, acc_v)

    nlo = wid * _NPW
    npw = jnp.minimum(_NPW, _N - nlo)
    bvec = bounds_v[wid, :]
    b0 = bvec[0]
    b1 = bvec[1]
    c_lo = b0 // _RC
    c_hi = (b1 + _RC - 1) // _RC

    def chunk_body(t, carry):
        base = _RC * t
        pltpu.sync_copy(act_hbm.at[pl.ds(base, _RC)], rows_v)
        pltpu.sync_copy(seg_hbm.at[pl.ds(base, _RC)], seg_v)

        def grp_body(g, carry2):
            sl = seg_v[pl.ds(_IN * g, _IN)] - nlo
            for l in range(_IN):
                s = sl[l]

                @pl.when((s >= 0) & (s < npw))
                def _():
                    for k in range(_H // _IN):
                        plsc.addupdate(
                            acc_v.at[s, pl.ds(_IN * k, _IN)],
                            rows_v[_IN * g + l, pl.ds(_IN * k, _IN)])

            return carry2

        lax.fori_loop(0, _RC // _IN, grp_body, 0)
        return carry

    lax.fori_loop(c_lo, c_hi, chunk_body, 0)

    @pl.when(wid < _NW - 1)
    def _():
        pltpu.sync_copy(acc_v, out_hbm.at[pl.ds(nlo, _NPW)])

    @pl.when(wid == _NW - 1)
    def _():
        pltpu.sync_copy(acc_v.at[pl.ds(0, _N - (_NW - 1) * _NPW)],
                        out_hbm.at[pl.ds(nlo, _N - (_NW - 1) * _NPW)])


def _segsum(act, seg_pad, bounds, zeros_blk):
    mesh = plsc.VectorSubcoreMesh(core_axis_name="c", subcore_axis_name="s",
                                  num_cores=_NC, num_subcores=_NS)
    kern = functools.partial(
        pl.kernel,
        out_type=jax.ShapeDtypeStruct((_N, _H), jnp.float32),
        mesh=mesh,
        scratch_types=[
            pltpu.VMEM((_RC, _H), jnp.float32),
            pltpu.VMEM((_RC,), jnp.int32),
            pltpu.VMEM((_NW, _IN), jnp.int32),
            pltpu.VMEM((_NPW, _H), jnp.float32),
        ],
    )(_segsum_body)
    return kern(act, seg_pad, bounds, zeros_blk)


_BN = 2000


def _tail_body(pooled_ref, ind_ref, w0_ref, b0_ref, w1_ref, b1_ref,
               wm_ref, bm_ref, o_ref):
    h0 = jnp.maximum(ind_ref[...] * w0_ref[...] + b0_ref[...], 0.0)
    factor = jnp.dot(h0, w1_ref[...],
                     preferred_element_type=jnp.float32) + b1_ref[...]
    y = jnp.maximum(pooled_ref[...] * factor, 0.0)
    z = jnp.dot(y, wm_ref[...], preferred_element_type=jnp.float32)
    o_ref[...] = jnp.maximum(z + bm_ref[...], 0.0)


def _tail(pooled, indegree, W0, b0, W1, b1, Wm, bm):
    return pl.pallas_call(
        _tail_body,
        grid=(_N // _BN,),
        in_specs=[
            pl.BlockSpec((_BN, _H), lambda i: (i, 0)),
            pl.BlockSpec((_BN, 1), lambda i: (i, 0)),
            pl.BlockSpec((1, 2 * _H), lambda i: (0, 0)),
            pl.BlockSpec((1, 2 * _H), lambda i: (0, 0)),
            pl.BlockSpec((2 * _H, _H), lambda i: (0, 0)),
            pl.BlockSpec((1, _H), lambda i: (0, 0)),
            pl.BlockSpec((_H, _H), lambda i: (0, 0)),
            pl.BlockSpec((1, _H), lambda i: (0, 0)),
        ],
        out_specs=pl.BlockSpec((_BN, _H), lambda i: (i, 0)),
        out_shape=jax.ShapeDtypeStruct((_N, _H), jnp.float32),
    )(pooled, indegree.reshape(_N, 1), W0, b0.reshape(1, 2 * _H),
      W1, b1.reshape(1, _H), Wm, bm.reshape(1, _H))


def kernel(node_feat, edge_feat, indegree, n2p_idx, e2p_idx, pool_seg,
           weight, bias, W0, b0, W1, b1, Wm, bm):
    n2pt = jnp.pad(n2p_idx.reshape(_P, _LL).T, ((0, 0), (0, _PP - _P)))
    e2pt = jnp.pad(e2p_idx.reshape(_P, _LL).T, ((0, 0), (0, _PP - _P)))
    w2 = jnp.transpose(weight, (2, 0, 1)).reshape(_LL * _IN, _H)

    perm = _gather_perm(node_feat, edge_feat, n2pt, e2pt)
    act = _matmul_act(perm, w2, bias)

    seg_pad = jnp.pad(pool_seg, (0, _PP - _P), constant_values=_N)
    starts = jnp.minimum(jnp.arange(33, dtype=jnp.int32) * _NPW, _N)
    bounds = jnp.searchsorted(seg_pad, starts, side="left").astype(jnp.int32)
    barr = jnp.stack([bounds[:32], bounds[1:33]], axis=1)
    barr = jnp.pad(barr, ((0, 0), (0, _IN - 2)))
    zeros_blk = jnp.zeros((_NPW, _H), jnp.float32)

    pooled = _segsum(act, seg_pad, barr, zeros_blk)
    out = _tail(pooled, indegree, W0, b0, W1, b1, Wm, bm)
    return (out, edge_feat)


# speedup vs baseline: 20.1262x; 1.4780x over previous
import functools

import jax
import jax.numpy as jnp
from jax import lax
from jax.experimental import pallas as pl
from jax.experimental.pallas import tpu as pltpu
from jax.experimental.pallas import tpu_sc as plsc

_N = 10000
_P = 100000
_LL = 16
_IN = 16
_H = 256

_NC = 2
_NS = 16
_NW = _NC * _NS

_CP = 64
_CF = _CP * _LL
_NST = _CF // 128
_NCHUNK = 1564
_PP = _NCHUNK * _CP
_NT = (_NCHUNK + _NW - 1) // _NW


def _gather_body(node_hbm, edge_hbm, n2pf_hbm, e2pf_hbm, out_hbm,
                 nidx_v, eidx_v, nbuf_v, ebuf_v, wide_v,
                 nsem0, nsem1, esem0, esem1, osem0, osem1):
    wid = lax.axis_index("s") * _NC + lax.axis_index("c")
    nsems = (nsem0, nsem1)
    esems = (esem0, esem1)
    osems = (osem0, osem1)

    def fire(t, s):
        c = wid + _NW * t

        @pl.when(c < _NCHUNK)
        def _():
            pltpu.sync_copy(n2pf_hbm.at[pl.ds(_NST * c, _NST)], nidx_v.at[s])
            pltpu.sync_copy(e2pf_hbm.at[pl.ds(_NST * c, _NST)], eidx_v.at[s])
            for j in range(_NST):
                pltpu.async_copy(node_hbm.at[nidx_v.at[s, j]],
                                 nbuf_v.at[s, pl.ds(128 * j, 128)], nsems[s])
                pltpu.async_copy(edge_hbm.at[eidx_v.at[s, j]],
                                 ebuf_v.at[s, pl.ds(128 * j, 128)], esems[s])

    def consume(u, s):
        c = wid + _NW * u

        @pl.when(c < _NCHUNK)
        def _():
            pltpu.make_async_copy(node_hbm.at[pl.ds(0, _CF)],
                                  nbuf_v.at[s], nsems[s]).wait()
            pltpu.make_async_copy(edge_hbm.at[pl.ds(0, _CF)],
                                  ebuf_v.at[s], esems[s]).wait()

            @pl.when(u >= 2)
            def _():
                cprev = wid + _NW * (u - 2)
                pltpu.make_async_copy(
                    wide_v.at[s],
                    out_hbm.at[pl.ds(_CP * cprev, _CP)], osems[s]).wait()

            def row_body(r, carry2):
                for a in range(_LL):
                    wide_v[s, r, pl.ds(_IN * a, _IN)] = (
                        nbuf_v[s, _LL * r + a, :] + ebuf_v[s, _LL * r + a, :])
                return carry2

            lax.fori_loop(0, _CP, row_body, 0)
            pltpu.async_copy(wide_v.at[s],
                             out_hbm.at[pl.ds(_CP * c, _CP)], osems[s])

    def outer(g, carry):
        for b in (0, 1):
            t = 2 * g + b
            fire(t, b)

            @pl.when(t >= 1)
            def _():
                consume(t - 1, 1 - b)

        return carry

    lax.fori_loop(0, (_NT + 2) // 2, outer, 0)
    pltpu.make_async_copy(wide_v.at[0],
                          out_hbm.at[pl.ds(0, _CP)], osem0).wait()
    pltpu.make_async_copy(wide_v.at[1],
                          out_hbm.at[pl.ds(0, _CP)], osem1).wait()


def _gather_perm(node_feat, edge_feat, n2pf, e2pf):
    mesh = plsc.VectorSubcoreMesh(core_axis_name="c", subcore_axis_name="s",
                                  num_cores=_NC, num_subcores=_NS)
    kern = functools.partial(
        pl.kernel,
        out_type=jax.ShapeDtypeStruct((_PP, _LL * _IN), jnp.float32),
        mesh=mesh,
        compiler_params=pltpu.CompilerParams(use_tc_tiling_on_sc=False),
        scratch_types=[
            pltpu.VMEM((2, _NST, 128), jnp.int32),
            pltpu.VMEM((2, _NST, 128), jnp.int32),
            pltpu.VMEM((2, _CF, _IN), jnp.float32),
            pltpu.VMEM((2, _CF, _IN), jnp.float32),
            pltpu.VMEM((2, _CP, _LL * _IN), jnp.float32),
            pltpu.SemaphoreType.DMA,
            pltpu.SemaphoreType.DMA,
            pltpu.SemaphoreType.DMA,
            pltpu.SemaphoreType.DMA,
            pltpu.SemaphoreType.DMA,
            pltpu.SemaphoreType.DMA,
        ],
    )(_gather_body)
    return kern(node_feat, edge_feat, n2pf, e2pf)


_BP = 2176


def _mm_body(p_ref, w_ref, b_ref, o_ref):
    acc = jnp.dot(p_ref[...], w_ref[...], preferred_element_type=jnp.float32)
    o_ref[...] = jnp.maximum(acc + b_ref[...], 0.0)


def _matmul_act(perm, w2, bias):
    return pl.pallas_call(
        _mm_body,
        grid=(_PP // _BP,),
        in_specs=[
            pl.BlockSpec((_BP, _LL * _IN), lambda i: (i, 0)),
            pl.BlockSpec((_LL * _IN, _H), lambda i: (0, 0)),
            pl.BlockSpec((1, _H), lambda i: (0, 0)),
        ],
        out_specs=pl.BlockSpec((_BP, _H), lambda i: (i, 0)),
        out_shape=jax.ShapeDtypeStruct((_PP, _H), jnp.float32),
    )(perm, w2, bias.reshape(1, _H))


_RC = 128
_RH = _RC // 2
_NPW = 320
_TRASH = _NPW


def _segsum_body(act_hbm, seg_hbm, bounds_hbm, zeros_hbm, out_hbm,
                 rows_v, seg_v, bounds_v, acc_v, hsem0, hsem1):
    wid = lax.axis_index("s") * _NC + lax.axis_index("c")
    hsems = (hsem0, hsem1)
    pltpu.sync_copy(bounds_hbm, bounds_v)
    pltpu.sync_copy(zeros_hbm, acc_v.at[pl.ds(0, _NPW)])

    nlo = wid * _NPW
    npw = jnp.minimum(_NPW, _N - nlo)
    bvec = bounds_v[wid, :]
    b0 = bvec[0]
    b1 = bvec[1]
    c_lo = b0 // _RC
    c_hi = (b1 + _RC - 1) // _RC

    def chunk_body(t, carry):
        base = _RC * t
        pltpu.sync_copy(seg_hbm.at[pl.ds(base, _RC)], seg_v)
        for h in (0, 1):
            pltpu.async_copy(act_hbm.at[pl.ds(base + _RH * h, _RH)],
                             rows_v.at[h], hsems[h])

        for h in (0, 1):
            pltpu.make_async_copy(act_hbm.at[pl.ds(0, _RH)],
                                  rows_v.at[h], hsems[h]).wait()

            def grp_body(g, carry2):
                sl = seg_v[pl.ds(_RH * h + _IN * g, _IN)] - nlo
                s_sel = jnp.where((sl >= 0) & (sl < npw), sl, _TRASH)
                for l in range(_IN):
                    s = s_sel[l]
                    for k in range(_H // _IN):
                        plsc.addupdate(
                            acc_v.at[s, pl.ds(_IN * k, _IN)],
                            rows_v[h, _IN * g + l, pl.ds(_IN * k, _IN)])
                return carry2

            lax.fori_loop(0, _RH // _IN, grp_body, 0)
        return carry

    lax.fori_loop(c_lo, c_hi, chunk_body, 0)

    @pl.when(wid < _NW - 1)
    def _():
        pltpu.sync_copy(acc_v.at[pl.ds(0, _NPW)], out_hbm.at[pl.ds(nlo, _NPW)])

    @pl.when(wid == _NW - 1)
    def _():
        pltpu.sync_copy(acc_v.at[pl.ds(0, _N - (_NW - 1) * _NPW)],
                        out_hbm.at[pl.ds(nlo, _N - (_NW - 1) * _NPW)])


def _segsum(act, seg_pad, bounds, zeros_blk):
    mesh = plsc.VectorSubcoreMesh(core_axis_name="c", subcore_axis_name="s",
                                  num_cores=_NC, num_subcores=_NS)
    kern = functools.partial(
        pl.kernel,
        out_type=jax.ShapeDtypeStruct((_N, _H), jnp.float32),
        mesh=mesh,
        scratch_types=[
            pltpu.VMEM((2, _RH, _H), jnp.float32),
            pltpu.VMEM((_RC,), jnp.int32),
            pltpu.VMEM((_NW, _IN), jnp.int32),
            pltpu.VMEM((_NPW + 8, _H), jnp.float32),
            pltpu.SemaphoreType.DMA,
            pltpu.SemaphoreType.DMA,
        ],
    )(_segsum_body)
    return kern(act, seg_pad, bounds, zeros_blk)


_BN = 2000


def _tail_body(pooled_ref, ind_ref, w0_ref, b0_ref, w1_ref, b1_ref,
               wm_ref, bm_ref, o_ref):
    h0 = jnp.maximum(ind_ref[...] * w0_ref[...] + b0_ref[...], 0.0)
    factor = jnp.dot(h0, w1_ref[...],
                     preferred_element_type=jnp.float32) + b1_ref[...]
    y = jnp.maximum(pooled_ref[...] * factor, 0.0)
    z = jnp.dot(y, wm_ref[...], preferred_element_type=jnp.float32)
    o_ref[...] = jnp.maximum(z + bm_ref[...], 0.0)


def _tail(pooled, indegree, W0, b0, W1, b1, Wm, bm):
    return pl.pallas_call(
        _tail_body,
        grid=(_N // _BN,),
        in_specs=[
            pl.BlockSpec((_BN, _H), lambda i: (i, 0)),
            pl.BlockSpec((_BN, 1), lambda i: (i, 0)),
            pl.BlockSpec((1, 2 * _H), lambda i: (0, 0)),
            pl.BlockSpec((1, 2 * _H), lambda i: (0, 0)),
            pl.BlockSpec((2 * _H, _H), lambda i: (0, 0)),
            pl.BlockSpec((1, _H), lambda i: (0, 0)),
            pl.BlockSpec((_H, _H), lambda i: (0, 0)),
            pl.BlockSpec((1, _H), lambda i: (0, 0)),
        ],
        out_specs=pl.BlockSpec((_BN, _H), lambda i: (i, 0)),
        out_shape=jax.ShapeDtypeStruct((_N, _H), jnp.float32),
    )(pooled, indegree.reshape(_N, 1), W0, b0.reshape(1, 2 * _H),
      W1, b1.reshape(1, _H), Wm, bm.reshape(1, _H))


def kernel(node_feat, edge_feat, indegree, n2p_idx, e2p_idx, pool_seg,
           weight, bias, W0, b0, W1, b1, Wm, bm):
    n2pf = jnp.pad(n2p_idx, (0, (_PP - _P) * _LL)).reshape(-1, 128)
    e2pf = jnp.pad(e2p_idx, (0, (_PP - _P) * _LL)).reshape(-1, 128)
    w2 = jnp.transpose(weight, (2, 0, 1)).reshape(_LL * _IN, _H)

    perm = _gather_perm(node_feat, edge_feat, n2pf, e2pf)
    act = _matmul_act(perm, w2, bias)

    seg_pad = jnp.pad(pool_seg, (0, _PP - _P), constant_values=_N)
    starts = jnp.minimum(jnp.arange(33, dtype=jnp.int32) * _NPW, _N)
    bounds = jnp.searchsorted(seg_pad, starts, side="left").astype(jnp.int32)
    barr = jnp.stack([bounds[:32], bounds[1:33]], axis=1)
    barr = jnp.pad(barr, ((0, 0), (0, _IN - 2)))
    zeros_blk = jnp.zeros((_NPW, _H), jnp.float32)

    pooled = _segsum(act, seg_pad, barr, zeros_blk)
    out = _tail(pooled, indegree, W0, b0, W1, b1, Wm, bm)
    return (out, edge_feat)


# speedup vs baseline: 22.5990x; 1.1229x over previous
import functools

import jax
import jax.numpy as jnp
from jax import lax
from jax.experimental import pallas as pl
from jax.experimental.pallas import tpu as pltpu
from jax.experimental.pallas import tpu_sc as plsc

_N = 10000
_P = 100000
_LL = 16
_IN = 16
_H = 256

_NC = 2
_NS = 16
_NW = _NC * _NS

_CP = 64
_CF = _CP * _LL
_NST = _CF // 128
_NCHUNK = 1564
_PP = _NCHUNK * _CP
_NT = (_NCHUNK + _NW - 1) // _NW


def _gather_body(node_hbm, edge_hbm, n2pf_hbm, e2pf_hbm, out_hbm,
                 nidx_v, eidx_v, nbuf_v, ebuf_v, wide_v,
                 nsem0, nsem1, esem0, esem1, osem0, osem1):
    wid = lax.axis_index("s") * _NC + lax.axis_index("c")
    nsems = (nsem0, nsem1)
    esems = (esem0, esem1)
    osems = (osem0, osem1)

    def fire(t, s):
        c = wid + _NW * t

        @pl.when(c < _NCHUNK)
        def _():
            pltpu.sync_copy(n2pf_hbm.at[pl.ds(_NST * c, _NST)], nidx_v.at[s])
            pltpu.sync_copy(e2pf_hbm.at[pl.ds(_NST * c, _NST)], eidx_v.at[s])
            for j in range(_NST):
                pltpu.async_copy(node_hbm.at[nidx_v.at[s, j]],
                                 nbuf_v.at[s, pl.ds(128 * j, 128)], nsems[s])
                pltpu.async_copy(edge_hbm.at[eidx_v.at[s, j]],
                                 ebuf_v.at[s, pl.ds(128 * j, 128)], esems[s])

    def consume(u, s):
        c = wid + _NW * u

        @pl.when(c < _NCHUNK)
        def _():
            pltpu.make_async_copy(node_hbm.at[pl.ds(0, _CF)],
                                  nbuf_v.at[s], nsems[s]).wait()
            pltpu.make_async_copy(edge_hbm.at[pl.ds(0, _CF)],
                                  ebuf_v.at[s], esems[s]).wait()

            @pl.when(u >= 2)
            def _():
                cprev = wid + _NW * (u - 2)
                pltpu.make_async_copy(
                    wide_v.at[s],
                    out_hbm.at[pl.ds(_CP * cprev, _CP)], osems[s]).wait()

            def grp_body(g, carry2):
                r0 = 8 * g
                nv = [nbuf_v[s, _LL * r0 + a, :] for a in range(_LL)]
                ev = [ebuf_v[s, _LL * r0 + a, :] for a in range(_LL)]
                for i in range(8):
                    nxt_n, nxt_e = [], []
                    for a in range(_LL):
                        wide_v[s, r0 + i, pl.ds(_IN * a, _IN)] = nv[a] + ev[a]
                        if i < 7:
                            nxt_n.append(nbuf_v[s, _LL * (r0 + i + 1) + a, :])
                            nxt_e.append(ebuf_v[s, _LL * (r0 + i + 1) + a, :])
                    nv, ev = nxt_n, nxt_e
                return carry2

            lax.fori_loop(0, _CP // 8, grp_body, 0)
            pltpu.async_copy(wide_v.at[s],
                             out_hbm.at[pl.ds(_CP * c, _CP)], osems[s])

    def outer(g, carry):
        for b in (0, 1):
            t = 2 * g + b
            fire(t, b)

            @pl.when(t >= 1)
            def _():
                consume(t - 1, 1 - b)

        return carry

    lax.fori_loop(0, (_NT + 2) // 2, outer, 0)
    pltpu.make_async_copy(wide_v.at[0],
                          out_hbm.at[pl.ds(0, _CP)], osem0).wait()
    pltpu.make_async_copy(wide_v.at[1],
                          out_hbm.at[pl.ds(0, _CP)], osem1).wait()


def _gather_perm(node_feat, edge_feat, n2pf, e2pf):
    mesh = plsc.VectorSubcoreMesh(core_axis_name="c", subcore_axis_name="s",
                                  num_cores=_NC, num_subcores=_NS)
    kern = functools.partial(
        pl.kernel,
        out_type=jax.ShapeDtypeStruct((_PP, _LL * _IN), jnp.float32),
        mesh=mesh,
        compiler_params=pltpu.CompilerParams(use_tc_tiling_on_sc=False),
        scratch_types=[
            pltpu.VMEM((2, _NST, 128), jnp.int32),
            pltpu.VMEM((2, _NST, 128), jnp.int32),
            pltpu.VMEM((2, _CF, _IN), jnp.float32),
            pltpu.VMEM((2, _CF, _IN), jnp.float32),
            pltpu.VMEM((2, _CP, _LL * _IN), jnp.float32),
            pltpu.SemaphoreType.DMA,
            pltpu.SemaphoreType.DMA,
            pltpu.SemaphoreType.DMA,
            pltpu.SemaphoreType.DMA,
            pltpu.SemaphoreType.DMA,
            pltpu.SemaphoreType.DMA,
        ],
    )(_gather_body)
    return kern(node_feat, edge_feat, n2pf, e2pf)


_BP = 2176


def _mm_body(p_ref, w_ref, b_ref, o_ref):
    acc = jnp.dot(p_ref[...], w_ref[...], preferred_element_type=jnp.float32)
    o_ref[...] = jnp.maximum(acc + b_ref[...], 0.0)


def _matmul_act(perm, w2, bias):
    return pl.pallas_call(
        _mm_body,
        grid=(_PP // _BP,),
        in_specs=[
            pl.BlockSpec((_BP, _LL * _IN), lambda i: (i, 0)),
            pl.BlockSpec((_LL * _IN, _H), lambda i: (0, 0)),
            pl.BlockSpec((1, _H), lambda i: (0, 0)),
        ],
        out_specs=pl.BlockSpec((_BP, _H), lambda i: (i, 0)),
        out_shape=jax.ShapeDtypeStruct((_PP, _H), jnp.float32),
    )(perm, w2, bias.reshape(1, _H))


_RC = 128
_RH = _RC // 2
_NPW = 320
_TRASH = _NPW


def _segsum_body(act_hbm, seg_hbm, bounds_hbm, zeros_hbm, out_hbm,
                 rows_v, seg_v, bounds_v, acc_v, hsem0, hsem1):
    wid = lax.axis_index("s") * _NC + lax.axis_index("c")
    hsems = (hsem0, hsem1)
    pltpu.sync_copy(bounds_hbm, bounds_v)
    pltpu.sync_copy(zeros_hbm, acc_v.at[pl.ds(0, _NPW)])

    nlo = wid * _NPW
    npw = jnp.minimum(_NPW, _N - nlo)
    bvec = bounds_v[wid, :]
    b0 = bvec[0]
    b1 = bvec[1]
    c_lo = b0 // _RC
    c_hi = (b1 + _RC - 1) // _RC

    def chunk_body(t, carry):
        base = _RC * t
        pltpu.sync_copy(seg_hbm.at[pl.ds(base, _RC)], seg_v)
        for h in (0, 1):
            pltpu.async_copy(act_hbm.at[pl.ds(base + _RH * h, _RH)],
                             rows_v.at[h], hsems[h])

        for h in (0, 1):
            pltpu.make_async_copy(act_hbm.at[pl.ds(0, _RH)],
                                  rows_v.at[h], hsems[h]).wait()

            def grp_body(g, carry2):
                sl = seg_v[pl.ds(_RH * h + _IN * g, _IN)] - nlo
                s_sel = jnp.where((sl >= 0) & (sl < npw), sl, _TRASH)
                svals = [s_sel[l] for l in range(_IN)]
                nk = _H // _IN
                vals = [rows_v[h, _IN * g, pl.ds(_IN * k, _IN)]
                        for k in range(nk)]
                for l in range(_IN):
                    nxt = []
                    for k in range(nk):
                        plsc.addupdate(
                            acc_v.at[svals[l], pl.ds(_IN * k, _IN)], vals[k])
                        if l < _IN - 1:
                            nxt.append(rows_v[h, _IN * g + l + 1,
                                              pl.ds(_IN * k, _IN)])
                    vals = nxt
                return carry2

            lax.fori_loop(0, _RH // _IN, grp_body, 0)
        return carry

    lax.fori_loop(c_lo, c_hi, chunk_body, 0)

    @pl.when(wid < _NW - 1)
    def _():
        pltpu.sync_copy(acc_v.at[pl.ds(0, _NPW)], out_hbm.at[pl.ds(nlo, _NPW)])

    @pl.when(wid == _NW - 1)
    def _():
        pltpu.sync_copy(acc_v.at[pl.ds(0, _N - (_NW - 1) * _NPW)],
                        out_hbm.at[pl.ds(nlo, _N - (_NW - 1) * _NPW)])


def _segsum(act, seg_pad, bounds, zeros_blk):
    mesh = plsc.VectorSubcoreMesh(core_axis_name="c", subcore_axis_name="s",
                                  num_cores=_NC, num_subcores=_NS)
    kern = functools.partial(
        pl.kernel,
        out_type=jax.ShapeDtypeStruct((_N, _H), jnp.float32),
        mesh=mesh,
        scratch_types=[
            pltpu.VMEM((2, _RH, _H), jnp.float32),
            pltpu.VMEM((_RC,), jnp.int32),
            pltpu.VMEM((_NW, _IN), jnp.int32),
            pltpu.VMEM((_NPW + 8, _H), jnp.float32),
            pltpu.SemaphoreType.DMA,
            pltpu.SemaphoreType.DMA,
        ],
    )(_segsum_body)
    return kern(act, seg_pad, bounds, zeros_blk)


_BN = 2000


def _tail_body(pooled_ref, ind_ref, w0_ref, b0_ref, w1_ref, b1_ref,
               wm_ref, bm_ref, o_ref):
    h0 = jnp.maximum(ind_ref[...] * w0_ref[...] + b0_ref[...], 0.0)
    factor = jnp.dot(h0, w1_ref[...],
                     preferred_element_type=jnp.float32) + b1_ref[...]
    y = jnp.maximum(pooled_ref[...] * factor, 0.0)
    z = jnp.dot(y, wm_ref[...], preferred_element_type=jnp.float32)
    o_ref[...] = jnp.maximum(z + bm_ref[...], 0.0)


def _tail(pooled, indegree, W0, b0, W1, b1, Wm, bm):
    return pl.pallas_call(
        _tail_body,
        grid=(_N // _BN,),
        in_specs=[
            pl.BlockSpec((_BN, _H), lambda i: (i, 0)),
            pl.BlockSpec((_BN, 1), lambda i: (i, 0)),
            pl.BlockSpec((1, 2 * _H), lambda i: (0, 0)),
            pl.BlockSpec((1, 2 * _H), lambda i: (0, 0)),
            pl.BlockSpec((2 * _H, _H), lambda i: (0, 0)),
            pl.BlockSpec((1, _H), lambda i: (0, 0)),
            pl.BlockSpec((_H, _H), lambda i: (0, 0)),
            pl.BlockSpec((1, _H), lambda i: (0, 0)),
        ],
        out_specs=pl.BlockSpec((_BN, _H), lambda i: (i, 0)),
        out_shape=jax.ShapeDtypeStruct((_N, _H), jnp.float32),
    )(pooled, indegree.reshape(_N, 1), W0, b0.reshape(1, 2 * _H),
      W1, b1.reshape(1, _H), Wm, bm.reshape(1, _H))


def kernel(node_feat, edge_feat, indegree, n2p_idx, e2p_idx, pool_seg,
           weight, bias, W0, b0, W1, b1, Wm, bm):
    n2pf = jnp.pad(n2p_idx, (0, (_PP - _P) * _LL)).reshape(-1, 128)
    e2pf = jnp.pad(e2p_idx, (0, (_PP - _P) * _LL)).reshape(-1, 128)
    w2 = jnp.transpose(weight, (2, 0, 1)).reshape(_LL * _IN, _H)

    perm = _gather_perm(node_feat, edge_feat, n2pf, e2pf)
    act = _matmul_act(perm, w2, bias)

    seg_pad = jnp.pad(pool_seg, (0, _PP - _P), constant_values=_N)
    starts = jnp.minimum(jnp.arange(33, dtype=jnp.int32) * _NPW, _N)
    bounds = jnp.searchsorted(seg_pad, starts, side="left").astype(jnp.int32)
    barr = jnp.stack([bounds[:32], bounds[1:33]], axis=1)
    barr = jnp.pad(barr, ((0, 0), (0, _IN - 2)))
    zeros_blk = jnp.zeros((_NPW, _H), jnp.float32)

    pooled = _segsum(act, seg_pad, barr, zeros_blk)
    out = _tail(pooled, indegree, W0, b0, W1, b1, Wm, bm)
    return (out, edge_feat)


# speedup vs baseline: 23.6525x; 1.0466x over previous
import functools

import jax
import jax.numpy as jnp
from jax import lax
from jax.experimental import pallas as pl
from jax.experimental.pallas import tpu as pltpu
from jax.experimental.pallas import tpu_sc as plsc

_N = 10000
_P = 100000
_LL = 16
_IN = 16
_H = 256

_NC = 2
_NS = 16
_NW = _NC * _NS

_CP = 64
_CF = _CP * _LL
_NST = _CF // 128
_NCHUNK = 1564
_PP = _NCHUNK * _CP
_NT = (_NCHUNK + _NW - 1) // _NW


def _gather_body(node_hbm, edge_hbm, n2pf_hbm, e2pf_hbm, out_hbm,
                 nidx_v, eidx_v, nbuf_v, ebuf_v, wide_v,
                 nsem0, nsem1, esem0, esem1, osem0, osem1):
    wid = lax.axis_index("s") * _NC + lax.axis_index("c")
    nsems = (nsem0, nsem1)
    esems = (esem0, esem1)
    osems = (osem0, osem1)

    def fire(t, s):
        c = wid + _NW * t

        @pl.when(c < _NCHUNK)
        def _():
            pltpu.sync_copy(n2pf_hbm.at[pl.ds(_NST * c, _NST)], nidx_v.at[s])
            pltpu.sync_copy(e2pf_hbm.at[pl.ds(_NST * c, _NST)], eidx_v.at[s])
            for j in range(_NST):
                pltpu.async_copy(node_hbm.at[nidx_v.at[s, j]],
                                 nbuf_v.at[s, pl.ds(128 * j, 128)], nsems[s])
                pltpu.async_copy(edge_hbm.at[eidx_v.at[s, j]],
                                 ebuf_v.at[s, pl.ds(128 * j, 128)], esems[s])

    def consume(u, s):
        c = wid + _NW * u

        @pl.when(c < _NCHUNK)
        def _():
            pltpu.make_async_copy(node_hbm.at[pl.ds(0, _CF)],
                                  nbuf_v.at[s], nsems[s]).wait()
            pltpu.make_async_copy(edge_hbm.at[pl.ds(0, _CF)],
                                  ebuf_v.at[s], esems[s]).wait()

            @pl.when(u >= 2)
            def _():
                cprev = wid + _NW * (u - 2)
                pltpu.make_async_copy(
                    wide_v.at[s],
                    out_hbm.at[pl.ds(_CP * cprev, _CP)], osems[s]).wait()

            def grp_body(g, carry2):
                r0 = 8 * g
                nv = [nbuf_v[s, _LL * r0 + a, :] for a in range(_LL)]
                ev = [ebuf_v[s, _LL * r0 + a, :] for a in range(_LL)]
                for i in range(8):
                    nxt_n, nxt_e = [], []
                    for a in range(_LL):
                        wide_v[s, r0 + i, pl.ds(_IN * a, _IN)] = nv[a] + ev[a]
                        if i < 7:
                            nxt_n.append(nbuf_v[s, _LL * (r0 + i + 1) + a, :])
                            nxt_e.append(ebuf_v[s, _LL * (r0 + i + 1) + a, :])
                    nv, ev = nxt_n, nxt_e
                return carry2

            lax.fori_loop(0, _CP // 8, grp_body, 0)
            pltpu.async_copy(wide_v.at[s],
                             out_hbm.at[pl.ds(_CP * c, _CP)], osems[s])

    def outer(g, carry):
        for b in (0, 1):
            t = 2 * g + b
            fire(t, b)

            @pl.when(t >= 1)
            def _():
                consume(t - 1, 1 - b)

        return carry

    lax.fori_loop(0, (_NT + 2) // 2, outer, 0)
    pltpu.make_async_copy(wide_v.at[0],
                          out_hbm.at[pl.ds(0, _CP)], osem0).wait()
    pltpu.make_async_copy(wide_v.at[1],
                          out_hbm.at[pl.ds(0, _CP)], osem1).wait()


def _gather_perm(node_feat, edge_feat, n2pf, e2pf):
    mesh = plsc.VectorSubcoreMesh(core_axis_name="c", subcore_axis_name="s",
                                  num_cores=_NC, num_subcores=_NS)
    kern = functools.partial(
        pl.kernel,
        out_type=jax.ShapeDtypeStruct((_PP, _LL * _IN), jnp.float32),
        mesh=mesh,
        compiler_params=pltpu.CompilerParams(use_tc_tiling_on_sc=False),
        scratch_types=[
            pltpu.VMEM((2, _NST, 128), jnp.int32),
            pltpu.VMEM((2, _NST, 128), jnp.int32),
            pltpu.VMEM((2, _CF, _IN), jnp.float32),
            pltpu.VMEM((2, _CF, _IN), jnp.float32),
            pltpu.VMEM((2, _CP, _LL * _IN), jnp.float32),
            pltpu.SemaphoreType.DMA,
            pltpu.SemaphoreType.DMA,
            pltpu.SemaphoreType.DMA,
            pltpu.SemaphoreType.DMA,
            pltpu.SemaphoreType.DMA,
            pltpu.SemaphoreType.DMA,
        ],
    )(_gather_body)
    return kern(node_feat, edge_feat, n2pf, e2pf)


_BP = 2176


def _mm_body(p_ref, w_ref, b_ref, o_ref):
    acc = jnp.dot(p_ref[...], w_ref[...], preferred_element_type=jnp.float32)
    o_ref[...] = jnp.maximum(acc + b_ref[...], 0.0)


def _matmul_act(perm, w2, bias):
    return pl.pallas_call(
        _mm_body,
        grid=(_PP // _BP,),
        in_specs=[
            pl.BlockSpec((_BP, _LL * _IN), lambda i: (i, 0)),
            pl.BlockSpec((_LL * _IN, _H), lambda i: (0, 0)),
            pl.BlockSpec((1, _H), lambda i: (0, 0)),
        ],
        out_specs=pl.BlockSpec((_BP, _H), lambda i: (i, 0)),
        out_shape=jax.ShapeDtypeStruct((_PP, _H), jnp.float32),
    )(perm, w2, bias.reshape(1, _H))


_RH = 64
_NPW = 320
_TRASH = _NPW


def _segsum_body(act_hbm, seg2_hbm, bounds_hbm, zeros_hbm, out_hbm,
                 rows_v, seg_v, bounds_v, acc_v, rsem0, rsem1, ssem0, ssem1):
    wid = lax.axis_index("s") * _NC + lax.axis_index("c")
    rsems = (rsem0, rsem1)
    ssems = (ssem0, ssem1)
    pltpu.sync_copy(bounds_hbm, bounds_v)
    pltpu.sync_copy(zeros_hbm, acc_v.at[pl.ds(0, _NPW)])

    nlo = wid * _NPW
    npw = jnp.minimum(_NPW, _N - nlo)
    bvec = bounds_v[wid, :]
    b0 = bvec[0]
    b1 = bvec[1]
    u_lo = (b0 // (4 * _RH)) * 4
    u_hi = (b1 + _RH - 1) // _RH

    def fire(v, vb):
        u = u_lo + v
        if vb % 2 == 0:
            sslot = (vb // 2) % 2

            @pl.when(u < u_hi)
            def _():
                pltpu.async_copy(seg2_hbm.at[u // 2], seg_v.at[sslot],
                                 ssems[sslot])

        @pl.when(u < u_hi)
        def _():
            pltpu.async_copy(act_hbm.at[pl.ds(_RH * u, _RH)],
                             rows_v.at[vb % 2], rsems[vb % 2])

    def consume(v, vb):
        u = u_lo + v
        s = vb % 2
        sslot = (vb // 2) % 2

        @pl.when(u < u_hi)
        def _():
            if vb % 2 == 0:
                pltpu.make_async_copy(seg2_hbm.at[0], seg_v.at[sslot],
                                      ssems[sslot]).wait()
            pltpu.make_async_copy(act_hbm.at[pl.ds(0, _RH)],
                                  rows_v.at[s], rsems[s]).wait()
            soff = _RH * (vb % 2)

            def grp_body(g, carry2):
                sl = seg_v[sslot, pl.ds(soff + _IN * g, _IN)] - nlo
                s_sel = jnp.where((sl >= 0) & (sl < npw), sl, _TRASH)
                svals = [s_sel[l] for l in range(_IN)]
                nk = _H // _IN
                vals = [rows_v[s, _IN * g, pl.ds(_IN * k, _IN)]
                        for k in range(nk)]
                for l in range(_IN):
                    nxt = []
                    for k in range(nk):
                        plsc.addupdate(
                            acc_v.at[svals[l], pl.ds(_IN * k, _IN)], vals[k])
                        if l < _IN - 1:
                            nxt.append(rows_v[s, _IN * g + l + 1,
                                              pl.ds(_IN * k, _IN)])
                    vals = nxt
                return carry2

            lax.fori_loop(0, _RH // _IN, grp_body, 0)

    fire(0, 0)

    def quad_body(q, carry):
        for b in range(4):
            v = 4 * q + b
            fire(v + 1, b + 1)
            consume(v, b)
        return carry

    lax.fori_loop(0, (u_hi - u_lo + 4) // 4, quad_body, 0)

    @pl.when(wid < _NW - 1)
    def _():
        pltpu.sync_copy(acc_v.at[pl.ds(0, _NPW)], out_hbm.at[pl.ds(nlo, _NPW)])

    @pl.when(wid == _NW - 1)
    def _():
        pltpu.sync_copy(acc_v.at[pl.ds(0, _N - (_NW - 1) * _NPW)],
                        out_hbm.at[pl.ds(nlo, _N - (_NW - 1) * _NPW)])


def _segsum(act, seg2d, bounds, zeros_blk):
    mesh = plsc.VectorSubcoreMesh(core_axis_name="c", subcore_axis_name="s",
                                  num_cores=_NC, num_subcores=_NS)
    kern = functools.partial(
        pl.kernel,
        out_type=jax.ShapeDtypeStruct((_N, _H), jnp.float32),
        mesh=mesh,
        scratch_types=[
            pltpu.VMEM((2, _RH, _H), jnp.float32),
            pltpu.VMEM((2, 128), jnp.int32),
            pltpu.VMEM((_NW, _IN), jnp.int32),
            pltpu.VMEM((_NPW + 8, _H), jnp.float32),
            pltpu.SemaphoreType.DMA,
            pltpu.SemaphoreType.DMA,
            pltpu.SemaphoreType.DMA,
            pltpu.SemaphoreType.DMA,
        ],
    )(_segsum_body)
    return kern(act, seg2d, bounds, zeros_blk)


_BN = 2000


def _tail_body(pooled_ref, ind_ref, w0_ref, b0_ref, w1_ref, b1_ref,
               wm_ref, bm_ref, o_ref):
    h0 = jnp.maximum(ind_ref[...] * w0_ref[...] + b0_ref[...], 0.0)
    factor = jnp.dot(h0, w1_ref[...],
                     preferred_element_type=jnp.float32) + b1_ref[...]
    y = jnp.maximum(pooled_ref[...] * factor, 0.0)
    z = jnp.dot(y, wm_ref[...], preferred_element_type=jnp.float32)
    o_ref[...] = jnp.maximum(z + bm_ref[...], 0.0)


def _tail(pooled, indegree, W0, b0, W1, b1, Wm, bm):
    return pl.pallas_call(
        _tail_body,
        grid=(_N // _BN,),
        in_specs=[
            pl.BlockSpec((_BN, _H), lambda i: (i, 0)),
            pl.BlockSpec((_BN, 1), lambda i: (i, 0)),
            pl.BlockSpec((1, 2 * _H), lambda i: (0, 0)),
            pl.BlockSpec((1, 2 * _H), lambda i: (0, 0)),
            pl.BlockSpec((2 * _H, _H), lambda i: (0, 0)),
            pl.BlockSpec((1, _H), lambda i: (0, 0)),
            pl.BlockSpec((_H, _H), lambda i: (0, 0)),
            pl.BlockSpec((1, _H), lambda i: (0, 0)),
        ],
        out_specs=pl.BlockSpec((_BN, _H), lambda i: (i, 0)),
        out_shape=jax.ShapeDtypeStruct((_N, _H), jnp.float32),
    )(pooled, indegree.reshape(_N, 1), W0, b0.reshape(1, 2 * _H),
      W1, b1.reshape(1, _H), Wm, bm.reshape(1, _H))


def kernel(node_feat, edge_feat, indegree, n2p_idx, e2p_idx, pool_seg,
           weight, bias, W0, b0, W1, b1, Wm, bm):
    n2pf = jnp.pad(n2p_idx, (0, (_PP - _P) * _LL)).reshape(-1, 128)
    e2pf = jnp.pad(e2p_idx, (0, (_PP - _P) * _LL)).reshape(-1, 128)
    w2 = jnp.transpose(weight, (2, 0, 1)).reshape(_LL * _IN, _H)

    perm = _gather_perm(node_feat, edge_feat, n2pf, e2pf)
    act = _matmul_act(perm, w2, bias)

    seg_pad = jnp.pad(pool_seg, (0, _PP - _P), constant_values=_N)
    starts = jnp.minimum(jnp.arange(33, dtype=jnp.int32) * _NPW, _N)
    bounds = jnp.searchsorted(seg_pad, starts, side="left").astype(jnp.int32)
    barr = jnp.stack([bounds[:32], bounds[1:33]], axis=1)
    barr = jnp.pad(barr, ((0, 0), (0, _IN - 2)))
    zeros_blk = jnp.zeros((_NPW, _H), jnp.float32)

    pooled = _segsum(act, seg_pad.reshape(_PP // 128, 128), barr, zeros_blk)
    out = _tail(pooled, indegree, W0, b0, W1, b1, Wm, bm)
    return (out, edge_feat)


# speedup vs baseline: 27.6853x; 1.1705x over previous
import functools

import jax
import jax.numpy as jnp
from jax import lax
from jax.experimental import pallas as pl
from jax.experimental.pallas import tpu as pltpu
from jax.experimental.pallas import tpu_sc as plsc

_N = 10000
_P = 100000
_LL = 16
_IN = 16
_H = 256

_NC = 2
_NS = 16
_NW = _NC * _NS

_CP = 64
_CF = _CP * _LL
_NST = _CF // 128
_NCHUNK = 1564
_PP = _NCHUNK * _CP
_NT = (_NCHUNK + _NW - 1) // _NW


def _gather_body(node_hbm, edge_hbm, n2pf_hbm, e2pf_hbm, out_hbm,
                 nidx_v, eidx_v, nbuf_v, ebuf_v, wide_v,
                 nsem0, nsem1, esem0, esem1, osem0, osem1):
    wid = lax.axis_index("s") * _NC + lax.axis_index("c")
    nsems = (nsem0, nsem1)
    esems = (esem0, esem1)
    osems = (osem0, osem1)

    def fire(t, s):
        c = wid + _NW * t

        @pl.when(c < _NCHUNK)
        def _():
            pltpu.sync_copy(n2pf_hbm.at[pl.ds(_NST * c, _NST)], nidx_v.at[s])
            pltpu.sync_copy(e2pf_hbm.at[pl.ds(_NST * c, _NST)], eidx_v.at[s])
            for j in range(_NST):
                pltpu.async_copy(node_hbm.at[nidx_v.at[s, j]],
                                 nbuf_v.at[s, pl.ds(128 * j, 128)], nsems[s])
                pltpu.async_copy(edge_hbm.at[eidx_v.at[s, j]],
                                 ebuf_v.at[s, pl.ds(128 * j, 128)], esems[s])

    def consume(u, s):
        c = wid + _NW * u

        @pl.when(c < _NCHUNK)
        def _():
            pltpu.make_async_copy(node_hbm.at[pl.ds(0, _CF)],
                                  nbuf_v.at[s], nsems[s]).wait()
            pltpu.make_async_copy(edge_hbm.at[pl.ds(0, _CF)],
                                  ebuf_v.at[s], esems[s]).wait()

            @pl.when(u >= 2)
            def _():
                cprev = wid + _NW * (u - 2)
                pltpu.make_async_copy(
                    wide_v.at[s, 0],
                    out_hbm.at[pl.ds(_CP * cprev, _CP)], osems[s]).wait()
                pltpu.make_async_copy(
                    wide_v.at[s, 1],
                    out_hbm.at[pl.ds(_PP + _CP * cprev, _CP)], osems[s]).wait()

            def grp_body(g, carry2):
                r0 = 8 * g
                nv = [nbuf_v[s, _LL * r0 + a, :] for a in range(_LL)]
                ev = [ebuf_v[s, _LL * r0 + a, :] for a in range(_LL)]
                for i in range(8):
                    nxt_n, nxt_e = [], []
                    for a in range(_LL):
                        wide_v[s, a // 8, r0 + i,
                               pl.ds(_IN * (a % 8), _IN)] = nv[a] + ev[a]
                        if i < 7:
                            nxt_n.append(nbuf_v[s, _LL * (r0 + i + 1) + a, :])
                            nxt_e.append(ebuf_v[s, _LL * (r0 + i + 1) + a, :])
                    nv, ev = nxt_n, nxt_e
                return carry2

            lax.fori_loop(0, _CP // 8, grp_body, 0)
            pltpu.async_copy(wide_v.at[s, 0],
                             out_hbm.at[pl.ds(_CP * c, _CP)], osems[s])
            pltpu.async_copy(wide_v.at[s, 1],
                             out_hbm.at[pl.ds(_PP + _CP * c, _CP)], osems[s])

    def outer(g, carry):
        for b in (0, 1):
            t = 2 * g + b
            fire(t, b)

            @pl.when(t >= 1)
            def _():
                consume(t - 1, 1 - b)

        return carry

    lax.fori_loop(0, (_NT + 2) // 2, outer, 0)
    for s in (0, 1):
        for hh in (0, 1):
            pltpu.make_async_copy(wide_v.at[s, hh],
                                  out_hbm.at[pl.ds(0, _CP)],
                                  osems[s]).wait()


def _gather_perm(node_feat, edge_feat, n2pf, e2pf):
    mesh = plsc.VectorSubcoreMesh(core_axis_name="c", subcore_axis_name="s",
                                  num_cores=_NC, num_subcores=_NS)
    kern = functools.partial(
        pl.kernel,
        out_type=jax.ShapeDtypeStruct((2 * _PP, 128), jnp.float32),
        mesh=mesh,
        compiler_params=pltpu.CompilerParams(use_tc_tiling_on_sc=False),
        scratch_types=[
            pltpu.VMEM((2, _NST, 128), jnp.int32),
            pltpu.VMEM((2, _NST, 128), jnp.int32),
            pltpu.VMEM((2, _CF, _IN), jnp.float32),
            pltpu.VMEM((2, _CF, _IN), jnp.float32),
            pltpu.VMEM((2, 2, _CP, 128), jnp.float32),
            pltpu.SemaphoreType.DMA,
            pltpu.SemaphoreType.DMA,
            pltpu.SemaphoreType.DMA,
            pltpu.SemaphoreType.DMA,
            pltpu.SemaphoreType.DMA,
            pltpu.SemaphoreType.DMA,
        ],
    )(_gather_body)
    return kern(node_feat, edge_feat, n2pf, e2pf)


_BP = 2176


def _mm_body(l_ref, r_ref, wt_ref, wb_ref, b_ref, o_ref):
    acc = jnp.dot(l_ref[...], wt_ref[...], preferred_element_type=jnp.float32)
    acc += jnp.dot(r_ref[...], wb_ref[...], preferred_element_type=jnp.float32)
    o_ref[...] = jnp.maximum(acc + b_ref[...], 0.0)


def _matmul_act(perm2, w2, bias):
    nblk = _PP // _BP
    return pl.pallas_call(
        _mm_body,
        grid=(nblk,),
        in_specs=[
            pl.BlockSpec((_BP, 128), lambda i: (i, 0)),
            pl.BlockSpec((_BP, 128), lambda i, n=nblk: (i + n, 0)),
            pl.BlockSpec((128, _H), lambda i: (0, 0)),
            pl.BlockSpec((128, _H), lambda i: (0, 0)),
            pl.BlockSpec((1, _H), lambda i: (0, 0)),
        ],
        out_specs=pl.BlockSpec((_BP, _H), lambda i: (i, 0)),
        out_shape=jax.ShapeDtypeStruct((_PP, _H), jnp.float32),
    )(perm2, perm2, w2[:128], w2[128:], bias.reshape(1, _H))


_RH = 64
_NPW = 320
_TRASH = _NPW


def _segsum_body(act_hbm, seg2_hbm, bounds_hbm, zeros_hbm, out_hbm,
                 rows_v, seg_v, bounds_v, acc_v, rsem0, rsem1, ssem0, ssem1):
    wid = lax.axis_index("s") * _NC + lax.axis_index("c")
    rsems = (rsem0, rsem1)
    ssems = (ssem0, ssem1)
    pltpu.sync_copy(bounds_hbm, bounds_v)
    pltpu.sync_copy(zeros_hbm, acc_v.at[pl.ds(0, _NPW)])

    nlo = wid * _NPW
    npw = jnp.minimum(_NPW, _N - nlo)
    bvec = bounds_v[wid, :]
    b0 = bvec[0]
    b1 = bvec[1]
    u_lo = (b0 // (4 * _RH)) * 4
    u_hi = (b1 + _RH - 1) // _RH

    def fire(v, vb):
        u = u_lo + v
        if vb % 2 == 0:
            sslot = (vb // 2) % 2

            @pl.when(u < u_hi)
            def _():
                pltpu.async_copy(seg2_hbm.at[u // 2], seg_v.at[sslot],
                                 ssems[sslot])

        @pl.when(u < u_hi)
        def _():
            pltpu.async_copy(act_hbm.at[pl.ds(_RH * u, _RH)],
                             rows_v.at[vb % 2], rsems[vb % 2])

    def consume(v, vb):
        u = u_lo + v
        s = vb % 2
        sslot = (vb // 2) % 2

        @pl.when(u < u_hi)
        def _():
            if vb % 2 == 0:
                pltpu.make_async_copy(seg2_hbm.at[0], seg_v.at[sslot],
                                      ssems[sslot]).wait()
            pltpu.make_async_copy(act_hbm.at[pl.ds(0, _RH)],
                                  rows_v.at[s], rsems[s]).wait()
            soff = _RH * (vb % 2)

            def grp_body(g, carry2):
                sl = seg_v[sslot, pl.ds(soff + _IN * g, _IN)] - nlo
                s_sel = jnp.where((sl >= 0) & (sl < npw), sl, _TRASH)
                svals = [s_sel[l] for l in range(_IN)]
                nk = _H // _IN
                vals = [rows_v[s, _IN * g, pl.ds(_IN * k, _IN)]
                        for k in range(nk)]
                for l in range(_IN):
                    nxt = []
                    for k in range(nk):
                        plsc.addupdate(
                            acc_v.at[svals[l], pl.ds(_IN * k, _IN)], vals[k])
                        if l < _IN - 1:
                            nxt.append(rows_v[s, _IN * g + l + 1,
                                              pl.ds(_IN * k, _IN)])
                    vals = nxt
                return carry2

            lax.fori_loop(0, _RH // _IN, grp_body, 0)

    fire(0, 0)

    def quad_body(q, carry):
        for b in range(4):
            v = 4 * q + b
            fire(v + 1, b + 1)
            consume(v, b)
        return carry

    lax.fori_loop(0, (u_hi - u_lo + 4) // 4, quad_body, 0)

    @pl.when(wid < _NW - 1)
    def _():
        pltpu.sync_copy(acc_v.at[pl.ds(0, _NPW)], out_hbm.at[pl.ds(nlo, _NPW)])

    @pl.when(wid == _NW - 1)
    def _():
        pltpu.sync_copy(acc_v.at[pl.ds(0, _N - (_NW - 1) * _NPW)],
                        out_hbm.at[pl.ds(nlo, _N - (_NW - 1) * _NPW)])


def _segsum(act, seg2d, bounds, zeros_blk):
    mesh = plsc.VectorSubcoreMesh(core_axis_name="c", subcore_axis_name="s",
                                  num_cores=_NC, num_subcores=_NS)
    kern = functools.partial(
        pl.kernel,
        out_type=jax.ShapeDtypeStruct((_N, _H), jnp.float32),
        mesh=mesh,
        scratch_types=[
            pltpu.VMEM((2, _RH, _H), jnp.float32),
            pltpu.VMEM((2, 128), jnp.int32),
            pltpu.VMEM((_NW, _IN), jnp.int32),
            pltpu.VMEM((_NPW + 8, _H), jnp.float32),
            pltpu.SemaphoreType.DMA,
            pltpu.SemaphoreType.DMA,
            pltpu.SemaphoreType.DMA,
            pltpu.SemaphoreType.DMA,
        ],
    )(_segsum_body)
    return kern(act, seg2d, bounds, zeros_blk)


_BN = 2000


def _tail_body(pooled_ref, ind_ref, w0_ref, b0_ref, w1_ref, b1_ref,
               wm_ref, bm_ref, o_ref):
    h0 = jnp.maximum(ind_ref[...] * w0_ref[...] + b0_ref[...], 0.0)
    factor = jnp.dot(h0, w1_ref[...],
                     preferred_element_type=jnp.float32) + b1_ref[...]
    y = jnp.maximum(pooled_ref[...] * factor, 0.0)
    z = jnp.dot(y, wm_ref[...], preferred_element_type=jnp.float32)
    o_ref[...] = jnp.maximum(z + bm_ref[...], 0.0)


def _tail(pooled, indegree, W0, b0, W1, b1, Wm, bm):
    return pl.pallas_call(
        _tail_body,
        grid=(_N // _BN,),
        in_specs=[
            pl.BlockSpec((_BN, _H), lambda i: (i, 0)),
            pl.BlockSpec((_BN, 1), lambda i: (i, 0)),
            pl.BlockSpec((1, 2 * _H), lambda i: (0, 0)),
            pl.BlockSpec((1, 2 * _H), lambda i: (0, 0)),
            pl.BlockSpec((2 * _H, _H), lambda i: (0, 0)),
            pl.BlockSpec((1, _H), lambda i: (0, 0)),
            pl.BlockSpec((_H, _H), lambda i: (0, 0)),
            pl.BlockSpec((1, _H), lambda i: (0, 0)),
        ],
        out_specs=pl.BlockSpec((_BN, _H), lambda i: (i, 0)),
        out_shape=jax.ShapeDtypeStruct((_N, _H), jnp.float32),
    )(pooled, indegree.reshape(_N, 1), W0, b0.reshape(1, 2 * _H),
      W1, b1.reshape(1, _H), Wm, bm.reshape(1, _H))


def kernel(node_feat, edge_feat, indegree, n2p_idx, e2p_idx, pool_seg,
           weight, bias, W0, b0, W1, b1, Wm, bm):
    n2pf = jnp.pad(n2p_idx, (0, (_PP - _P) * _LL)).reshape(-1, 128)
    e2pf = jnp.pad(e2p_idx, (0, (_PP - _P) * _LL)).reshape(-1, 128)
    w2 = jnp.transpose(weight, (2, 0, 1)).reshape(_LL * _IN, _H)

    perm = _gather_perm(node_feat, edge_feat, n2pf, e2pf)
    act = _matmul_act(perm, w2, bias)

    seg_pad = jnp.pad(pool_seg, (0, _PP - _P), constant_values=_N)
    starts = jnp.minimum(jnp.arange(33, dtype=jnp.int32) * _NPW, _N)
    bounds = jnp.searchsorted(seg_pad, starts, side="left").astype(jnp.int32)
    barr = jnp.stack([bounds[:32], bounds[1:33]], axis=1)
    barr = jnp.pad(barr, ((0, 0), (0, _IN - 2)))
    zeros_blk = jnp.zeros((_NPW, _H), jnp.float32)

    pooled = _segsum(act, seg_pad.reshape(_PP // 128, 128), barr, zeros_blk)
    out = _tail(pooled, indegree, W0, b0, W1, b1, Wm, bm)
    return (out, edge_feat)


# speedup vs baseline: 28.0407x; 1.0128x over previous
import functools

import jax
import jax.numpy as jnp
from jax import lax
from jax.experimental import pallas as pl
from jax.experimental.pallas import tpu as pltpu
from jax.experimental.pallas import tpu_sc as plsc

_N = 10000
_P = 100000
_LL = 16
_IN = 16
_H = 256

_NC = 2
_NS = 16
_NW = _NC * _NS

_CP = 64
_CF = _CP * _LL
_NST = _CF // 128
_NCHUNK = 1564
_PP = _NCHUNK * _CP
_NT = (_NCHUNK + _NW - 1) // _NW


def _gather_body(node_hbm, edge_hbm, n2pf_hbm, e2pf_hbm, out_hbm,
                 nidx_v, eidx_v, nbuf_v, ebuf_v, wide_v,
                 nsem0, nsem1, esem0, esem1, osem0, osem1):
    wid = lax.axis_index("s") * _NC + lax.axis_index("c")
    nsems = (nsem0, nsem1)
    esems = (esem0, esem1)
    osems = (osem0, osem1)

    def fire(t, s):
        c = wid + _NW * t

        @pl.when(c < _NCHUNK)
        def _():
            pltpu.sync_copy(n2pf_hbm.at[pl.ds(_NST * c, _NST)], nidx_v.at[s])
            pltpu.sync_copy(e2pf_hbm.at[pl.ds(_NST * c, _NST)], eidx_v.at[s])
            for j in range(_NST):
                pltpu.async_copy(node_hbm.at[nidx_v.at[s, j]],
                                 nbuf_v.at[s, pl.ds(128 * j, 128)], nsems[s])
                pltpu.async_copy(edge_hbm.at[eidx_v.at[s, j]],
                                 ebuf_v.at[s, pl.ds(128 * j, 128)], esems[s])

    def consume(u, s):
        c = wid + _NW * u

        @pl.when(c < _NCHUNK)
        def _():
            pltpu.make_async_copy(node_hbm.at[pl.ds(0, _CF)],
                                  nbuf_v.at[s], nsems[s]).wait()
            pltpu.make_async_copy(edge_hbm.at[pl.ds(0, _CF)],
                                  ebuf_v.at[s], esems[s]).wait()

            @pl.when(u >= 2)
            def _():
                cprev = wid + _NW * (u - 2)
                pltpu.make_async_copy(
                    wide_v.at[s, 0],
                    out_hbm.at[pl.ds(_CP * cprev, _CP)], osems[s]).wait()
                pltpu.make_async_copy(
                    wide_v.at[s, 1],
                    out_hbm.at[pl.ds(_PP + _CP * cprev, _CP)], osems[s]).wait()

            def grp_body(g, carry2):
                r0 = 8 * g
                nv = [nbuf_v[s, _LL * r0 + a, :] for a in range(_LL)]
                ev = [ebuf_v[s, _LL * r0 + a, :] for a in range(_LL)]
                for i in range(8):
                    nxt_n, nxt_e = [], []
                    for a in range(_LL):
                        wide_v[s, a // 8, r0 + i,
                               pl.ds(_IN * (a % 8), _IN)] = nv[a] + ev[a]
                        if i < 7:
                            nxt_n.append(nbuf_v[s, _LL * (r0 + i + 1) + a, :])
                            nxt_e.append(ebuf_v[s, _LL * (r0 + i + 1) + a, :])
                    nv, ev = nxt_n, nxt_e
                return carry2

            lax.fori_loop(0, _CP // 8, grp_body, 0)
            pltpu.async_copy(wide_v.at[s, 0],
                             out_hbm.at[pl.ds(_CP * c, _CP)], osems[s])
            pltpu.async_copy(wide_v.at[s, 1],
                             out_hbm.at[pl.ds(_PP + _CP * c, _CP)], osems[s])

    def outer(g, carry):
        for b in (0, 1):
            t = 2 * g + b
            fire(t, b)

            @pl.when(t >= 1)
            def _():
                consume(t - 1, 1 - b)

        return carry

    lax.fori_loop(0, (_NT + 2) // 2, outer, 0)
    for s in (0, 1):
        for hh in (0, 1):
            pltpu.make_async_copy(wide_v.at[s, hh],
                                  out_hbm.at[pl.ds(0, _CP)],
                                  osems[s]).wait()


def _gather_perm(node_feat, edge_feat, n2pf, e2pf):
    mesh = plsc.VectorSubcoreMesh(core_axis_name="c", subcore_axis_name="s",
                                  num_cores=_NC, num_subcores=_NS)
    kern = functools.partial(
        pl.kernel,
        out_type=jax.ShapeDtypeStruct((2 * _PP, 128), jnp.float32),
        mesh=mesh,
        compiler_params=pltpu.CompilerParams(use_tc_tiling_on_sc=False),
        scratch_types=[
            pltpu.VMEM((2, _NST, 128), jnp.int32),
            pltpu.VMEM((2, _NST, 128), jnp.int32),
            pltpu.VMEM((2, _CF, _IN), jnp.float32),
            pltpu.VMEM((2, _CF, _IN), jnp.float32),
            pltpu.VMEM((2, 2, _CP, 128), jnp.float32),
            pltpu.SemaphoreType.DMA,
            pltpu.SemaphoreType.DMA,
            pltpu.SemaphoreType.DMA,
            pltpu.SemaphoreType.DMA,
            pltpu.SemaphoreType.DMA,
            pltpu.SemaphoreType.DMA,
        ],
    )(_gather_body)
    return kern(node_feat, edge_feat, n2pf, e2pf)


_BP = 2176


def _mm_body(l_ref, r_ref, wt_ref, wb_ref, b_ref, o_ref):
    acc = jnp.dot(l_ref[...], wt_ref[...], preferred_element_type=jnp.float32)
    acc += jnp.dot(r_ref[...], wb_ref[...], preferred_element_type=jnp.float32)
    o_ref[...] = jnp.maximum(acc + b_ref[...], 0.0)


def _matmul_act(perm2, w2, bias):
    nblk = _PP // _BP
    return pl.pallas_call(
        _mm_body,
        grid=(nblk,),
        in_specs=[
            pl.BlockSpec((_BP, 128), lambda i: (i, 0)),
            pl.BlockSpec((_BP, 128), lambda i, n=nblk: (i + n, 0)),
            pl.BlockSpec((128, _H), lambda i: (0, 0)),
            pl.BlockSpec((128, _H), lambda i: (0, 0)),
            pl.BlockSpec((1, _H), lambda i: (0, 0)),
        ],
        out_specs=pl.BlockSpec((_BP, _H), lambda i: (i, 0)),
        out_shape=jax.ShapeDtypeStruct((_PP, _H), jnp.float32),
    )(perm2, perm2, w2[:128], w2[128:], bias.reshape(1, _H))


_RH = 32
_NPW = 320
_TRASH = _NPW


def _segsum_body(act_hbm, seg2_hbm, bounds_hbm, zeros_hbm, out_hbm,
                 rows_v, seg_v, bounds_v, acc_v,
                 rsem0, rsem1, rsem2, rsem3, ssem0, ssem1):
    wid = lax.axis_index("s") * _NC + lax.axis_index("c")
    rsems = (rsem0, rsem1, rsem2, rsem3)
    ssems = (ssem0, ssem1)
    pltpu.sync_copy(bounds_hbm, bounds_v)
    pltpu.sync_copy(zeros_hbm, acc_v.at[pl.ds(0, _NPW)])

    nlo = wid * _NPW
    npw = jnp.minimum(_NPW, _N - nlo)
    bvec = bounds_v[wid, :]
    b0 = bvec[0]
    b1 = bvec[1]
    u_lo = (b0 // (8 * _RH)) * 8
    u_hi = (b1 + _RH - 1) // _RH

    def fire(v, vb):
        u = u_lo + v
        if vb % 4 == 0:
            sslot = (vb // 4) % 2

            @pl.when(u < u_hi)
            def _():
                pltpu.async_copy(seg2_hbm.at[u // 4], seg_v.at[sslot],
                                 ssems[sslot])

        @pl.when(u < u_hi)
        def _():
            pltpu.async_copy(act_hbm.at[pl.ds(_RH * u, _RH)],
                             rows_v.at[vb % 4], rsems[vb % 4])

    def consume(v, vb):
        u = u_lo + v
        s = vb % 4
        sslot = (vb // 4) % 2

        @pl.when(u < u_hi)
        def _():
            if vb % 4 == 0:
                pltpu.make_async_copy(seg2_hbm.at[0], seg_v.at[sslot],
                                      ssems[sslot]).wait()
            pltpu.make_async_copy(act_hbm.at[pl.ds(0, _RH)],
                                  rows_v.at[s], rsems[s]).wait()
            soff = _RH * (vb % 4)

            def grp_body(g, carry2):
                sl = seg_v[sslot, pl.ds(soff + _IN * g, _IN)] - nlo
                s_sel = jnp.where((sl >= 0) & (sl < npw), sl, _TRASH)
                svals = [s_sel[l] for l in range(_IN)]
                nk = _H // _IN
                vals = [rows_v[s, _IN * g, pl.ds(_IN * k, _IN)]
                        for k in range(nk)]
                for l in range(_IN):
                    nxt = []
                    for k in range(nk):
                        plsc.addupdate(
                            acc_v.at[svals[l], pl.ds(_IN * k, _IN)], vals[k])
                        if l < _IN - 1:
                            nxt.append(rows_v[s, _IN * g + l + 1,
                                              pl.ds(_IN * k, _IN)])
                    vals = nxt
                return carry2

            lax.fori_loop(0, _RH // _IN, grp_body, 0)

    for pv in range(3):
        fire(pv, pv)

    def oct_body(q, carry):
        for b in range(8):
            v = 8 * q + b
            fire(v + 3, b + 3)
            consume(v, b)
        return carry

    lax.fori_loop(0, (u_hi - u_lo + 8) // 8, oct_body, 0)

    @pl.when(wid < _NW - 1)
    def _():
        pltpu.sync_copy(acc_v.at[pl.ds(0, _NPW)], out_hbm.at[pl.ds(nlo, _NPW)])

    @pl.when(wid == _NW - 1)
    def _():
        pltpu.sync_copy(acc_v.at[pl.ds(0, _N - (_NW - 1) * _NPW)],
                        out_hbm.at[pl.ds(nlo, _N - (_NW - 1) * _NPW)])


def _segsum(act, seg2d, bounds, zeros_blk):
    mesh = plsc.VectorSubcoreMesh(core_axis_name="c", subcore_axis_name="s",
                                  num_cores=_NC, num_subcores=_NS)
    kern = functools.partial(
        pl.kernel,
        out_type=jax.ShapeDtypeStruct((_N, _H), jnp.float32),
        mesh=mesh,
        scratch_types=[
            pltpu.VMEM((4, _RH, _H), jnp.float32),
            pltpu.VMEM((2, 128), jnp.int32),
            pltpu.VMEM((_NW, _IN), jnp.int32),
            pltpu.VMEM((_NPW + 8, _H), jnp.float32),
            pltpu.SemaphoreType.DMA,
            pltpu.SemaphoreType.DMA,
            pltpu.SemaphoreType.DMA,
            pltpu.SemaphoreType.DMA,
            pltpu.SemaphoreType.DMA,
            pltpu.SemaphoreType.DMA,
        ],
    )(_segsum_body)
    return kern(act, seg2d, bounds, zeros_blk)


_BN = 2000


def _tail_body(pooled_ref, ind_ref, w0_ref, b0_ref, w1_ref, b1_ref,
               wm_ref, bm_ref, o_ref):
    h0 = jnp.maximum(ind_ref[...] * w0_ref[...] + b0_ref[...], 0.0)
    factor = jnp.dot(h0, w1_ref[...],
                     preferred_element_type=jnp.float32) + b1_ref[...]
    y = jnp.maximum(pooled_ref[...] * factor, 0.0)
    z = jnp.dot(y, wm_ref[...], preferred_element_type=jnp.float32)
    o_ref[...] = jnp.maximum(z + bm_ref[...], 0.0)


def _tail(pooled, indegree, W0, b0, W1, b1, Wm, bm):
    return pl.pallas_call(
        _tail_body,
        grid=(_N // _BN,),
        in_specs=[
            pl.BlockSpec((_BN, _H), lambda i: (i, 0)),
            pl.BlockSpec((_BN, 1), lambda i: (i, 0)),
            pl.BlockSpec((1, 2 * _H), lambda i: (0, 0)),
            pl.BlockSpec((1, 2 * _H), lambda i: (0, 0)),
            pl.BlockSpec((2 * _H, _H), lambda i: (0, 0)),
            pl.BlockSpec((1, _H), lambda i: (0, 0)),
            pl.BlockSpec((_H, _H), lambda i: (0, 0)),
            pl.BlockSpec((1, _H), lambda i: (0, 0)),
        ],
        out_specs=pl.BlockSpec((_BN, _H), lambda i: (i, 0)),
        out_shape=jax.ShapeDtypeStruct((_N, _H), jnp.float32),
    )(pooled, indegree.reshape(_N, 1), W0, b0.reshape(1, 2 * _H),
      W1, b1.reshape(1, _H), Wm, bm.reshape(1, _H))


def kernel(node_feat, edge_feat, indegree, n2p_idx, e2p_idx, pool_seg,
           weight, bias, W0, b0, W1, b1, Wm, bm):
    n2pf = jnp.pad(n2p_idx, (0, (_PP - _P) * _LL)).reshape(-1, 128)
    e2pf = jnp.pad(e2p_idx, (0, (_PP - _P) * _LL)).reshape(-1, 128)
    w2 = jnp.transpose(weight, (2, 0, 1)).reshape(_LL * _IN, _H)

    perm = _gather_perm(node_feat, edge_feat, n2pf, e2pf)
    act = _matmul_act(perm, w2, bias)

    seg_pad = jnp.pad(pool_seg, (0, _PP - _P), constant_values=_N)
    starts = jnp.minimum(jnp.arange(33, dtype=jnp.int32) * _NPW, _N)
    bounds = jnp.searchsorted(seg_pad, starts, side="left").astype(jnp.int32)
    barr = jnp.stack([bounds[:32], bounds[1:33]], axis=1)
    barr = jnp.pad(barr, ((0, 0), (0, _IN - 2)))
    zeros_blk = jnp.zeros((_NPW, _H), jnp.float32)

    pooled = _segsum(act, seg_pad.reshape(_PP // 128, 128), barr, zeros_blk)
    out = _tail(pooled, indegree, W0, b0, W1, b1, Wm, bm)
    return (out, edge_feat)
